# Initial kernel scaffold; baseline (speedup 1.0000x reference)
import jax
import jax.numpy as jnp
from jax.experimental import pallas as pl


def kernel(src, flow):
    raise NotImplementedError("write your pallas kernel here")


# speedup vs baseline: 1.1812x; 1.1812x over previous
import functools

import jax
import jax.numpy as jnp
from jax import lax
from jax.experimental import pallas as pl
from jax.experimental.pallas import tpu as pltpu
from jax.experimental.pallas import tpu_sc as plsc

_B, _H, _W = 8, 512, 512
_P = _B * _H * _W
_NC, _NS, _L = 2, 16, 16
_NW = _NC * _NS
_PIX_PER_W = _P // _NW
_CHUNK = 4096
_NCHUNK = _PIX_PER_W // _CHUNK
_GB = 128
_NGB = _CHUNK // _GB


def _floor_parts(g):
    t = g.astype(jnp.int32)
    tf = t.astype(jnp.float32)
    f = jnp.where(tf > g, tf - 1.0, tf)
    return f.astype(jnp.int32), g - f


def _sc_body(ta, tb, tc_, td, fx_hbm, fy_hbm, out_hbm,
             fxv, fyv, idxv, pav, pbv, pcv, pdv,
             wav, wbv, wcv, wdv, outv, sem):
    wid = lax.axis_index("s") * _NC + lax.axis_index("c")
    iota = lax.iota(jnp.int32, _L)

    def chunk_body(c, _):
        base = wid * _PIX_PER_W + c * _CHUNK
        pltpu.sync_copy(fx_hbm.at[pl.ds(base, _CHUNK)], fxv)
        pltpu.sync_copy(fy_hbm.at[pl.ds(base, _CHUNK)], fyv)

        def idx_body(i, _):
            off = i * _L
            p = base + off + iota
            x = p & (_W - 1)
            y = (p >> 9) & (_H - 1)
            gx = x.astype(jnp.float32) + fxv[pl.ds(off, _L)]
            gy = y.astype(jnp.float32) + fyv[pl.ds(off, _L)]
            x0, fxr = _floor_parts(gx)
            y0, fyr = _floor_parts(gy)
            exr = 1.0 - fxr
            eyr = 1.0 - fyr
            wa = exr * eyr
            wb = fxr * eyr
            wc = exr * fyr
            wd = fxr * fyr
            zero = jnp.zeros_like(wa)
            mx = gx < 0.0
            wa = jnp.where(mx, wa + wb, wa)
            wb = jnp.where(mx, zero, wb)
            wc = jnp.where(mx, wc + wd, wc)
            wd = jnp.where(mx, zero, wd)
            my = gy < 0.0
            wa = jnp.where(my, wa + wc, wa)
            wc = jnp.where(my, zero, wc)
            wb = jnp.where(my, wb + wd, wb)
            wd = jnp.where(my, zero, wd)
            x0c = jnp.minimum(jnp.maximum(x0, 0), _W - 1)
            y0c = jnp.minimum(jnp.maximum(y0, 0), _H - 1)
            gidx = ((p >> 18) << 18) + (y0c << 9) + x0c
            idxv[pl.ds(off, _L)] = gidx
            wav[pl.ds(off, _L)] = wa
            wbv[pl.ds(off, _L)] = wb
            wcv[pl.ds(off, _L)] = wc
            wdv[pl.ds(off, _L)] = wd
            return 0

        lax.fori_loop(0, _CHUNK // _L, idx_body, 0)

        def fire(j, _):
            sl = pl.ds(j * _GB, _GB)
            isl = idxv.at[sl]
            pltpu.async_copy(ta.at[isl], pav.at[sl], sem)
            pltpu.async_copy(tb.at[isl], pbv.at[sl], sem)
            pltpu.async_copy(tc_.at[isl], pcv.at[sl], sem)
            pltpu.async_copy(td.at[isl], pdv.at[sl], sem)
            return 0

        lax.fori_loop(0, _NGB, fire, 0)
        pltpu.make_async_copy(ta.at[pl.ds(0, _CHUNK)], pav, sem).wait()
        pltpu.make_async_copy(ta.at[pl.ds(0, _CHUNK)], pbv, sem).wait()
        pltpu.make_async_copy(ta.at[pl.ds(0, _CHUNK)], pcv, sem).wait()
        pltpu.make_async_copy(ta.at[pl.ds(0, _CHUNK)], pdv, sem).wait()

        def blend_body(i, _):
            off = i * _L
            sl = pl.ds(off, _L)
            o = (wav[sl] * pav[sl] + wbv[sl] * pbv[sl]
                 + wcv[sl] * pcv[sl] + wdv[sl] * pdv[sl])
            outv[sl] = o
            return 0

        lax.fori_loop(0, _CHUNK // _L, blend_body, 0)
        pltpu.sync_copy(outv, out_hbm.at[pl.ds(base, _CHUNK)])
        return 0

    lax.fori_loop(0, _NCHUNK, chunk_body, 0)


_sc_call = functools.partial(
    pl.kernel,
    out_type=jax.ShapeDtypeStruct((_P,), jnp.float32),
    mesh=plsc.VectorSubcoreMesh(core_axis_name="c", subcore_axis_name="s",
                                num_cores=_NC, num_subcores=_NS),
    scratch_types=[
        pltpu.VMEM((_CHUNK,), jnp.float32),
        pltpu.VMEM((_CHUNK,), jnp.float32),
        pltpu.VMEM((_CHUNK,), jnp.int32),
        pltpu.VMEM((_CHUNK,), jnp.float32),
        pltpu.VMEM((_CHUNK,), jnp.float32),
        pltpu.VMEM((_CHUNK,), jnp.float32),
        pltpu.VMEM((_CHUNK,), jnp.float32),
        pltpu.VMEM((_CHUNK,), jnp.float32),
        pltpu.VMEM((_CHUNK,), jnp.float32),
        pltpu.VMEM((_CHUNK,), jnp.float32),
        pltpu.VMEM((_CHUNK,), jnp.float32),
        pltpu.VMEM((_CHUNK,), jnp.float32),
        pltpu.SemaphoreType.DMA,
    ],
)(_sc_body)


def kernel(src, flow):
    s = src[..., 0]
    sx = jnp.concatenate([s[:, :, 1:], s[:, :, -1:]], axis=2)
    sy = jnp.concatenate([s[:, 1:, :], s[:, -1:, :]], axis=1)
    sxy = jnp.concatenate([sx[:, 1:, :], sx[:, -1:, :]], axis=1)
    fx = flow[..., 0].reshape(_P)
    fy = flow[..., 1].reshape(_P)
    out = _sc_call(s.reshape(_P), sx.reshape(_P), sy.reshape(_P),
                   sxy.reshape(_P), fx, fy)
    return out.reshape(_B, _H, _W, 1)
